# baseline (speedup 1.0000x reference)
import functools

import jax
import jax.numpy as jnp
from jax import lax
from jax.experimental import pallas as pl
from jax.experimental.pallas import tpu as pltpu
from jax.experimental.pallas import tpu_sc as plsc

_CHUNK = 512


@functools.lru_cache(maxsize=None)
def _build_gather(N, M, D):
    info = plsc.get_sparse_core_info()
    num_workers = info.num_cores * info.num_subcores
    rows_per_w = N // num_workers
    C = min(_CHUNK, rows_per_w)
    n_chunks = rows_per_w // C
    assert n_chunks * C == rows_per_w

    mesh = plsc.VectorSubcoreMesh(core_axis_name="c", subcore_axis_name="s")

    @functools.partial(
        pl.kernel,
        mesh=mesh,
        out_type=jax.ShapeDtypeStruct((N, D), jnp.float32),
        scratch_types=[
            pltpu.VMEM((C,), jnp.int32),
            pltpu.VMEM((C, D), jnp.float32),
            pltpu.SemaphoreType.DMA,
        ],
        compiler_params=pltpu.CompilerParams(use_tc_tiling_on_sc=False),
    )
    def gather_kernel(tbl_hbm, ids_hbm, out_hbm, idx_v, rows_v, sem):
        wid = lax.axis_index("s") * info.num_cores + lax.axis_index("c")
        base = wid * rows_per_w

        def chunk_body(j, carry):
            off = base + j * C
            pltpu.sync_copy(ids_hbm.at[pl.ds(off, C)], idx_v)
            pltpu.async_copy(tbl_hbm.at[idx_v], rows_v, sem).wait()
            pltpu.sync_copy(rows_v, out_hbm.at[pl.ds(off, C)])
            return carry

        lax.fori_loop(0, n_chunks, chunk_body, 0)

    return gather_kernel


@functools.lru_cache(maxsize=None)
def _build_pack(N, D):
    info = plsc.get_sparse_core_info()
    L = info.num_lanes
    G = 128 // D
    num_workers = info.num_cores * info.num_subcores
    rows_per_w = N // num_workers
    C = min(_CHUNK, rows_per_w)
    n_chunks = rows_per_w // C
    assert n_chunks * C == rows_per_w

    mesh = plsc.VectorSubcoreMesh(core_axis_name="c", subcore_axis_name="s")

    @functools.partial(
        pl.kernel,
        mesh=mesh,
        out_type=jax.ShapeDtypeStruct((N, D), jnp.float32),
        scratch_types=[
            pltpu.VMEM((C // G, 128), jnp.float32),
            pltpu.VMEM((C, D), jnp.float32),
            pltpu.SemaphoreType.DMA,
        ],
    )
    def pack_kernel(y_hbm, out_hbm, rows_v, out_v, sem):
        wid = lax.axis_index("s") * info.num_cores + lax.axis_index("c")
        base = wid * rows_per_w

        base4 = wid * (rows_per_w // G)

        def chunk_body(j, carry):
            off = pl.multiple_of(base + j * C, 8)
            off4 = pl.multiple_of(base4 + j * (C // G), 8)
            pltpu.sync_copy(y_hbm.at[pl.ds(off4, C // G)], rows_v)

            def unpack_body(t, carry):
                for g in range(G):
                    i = t * G + g
                    for h in range(D // L):
                        out_v[i, pl.ds(h * L, L)] = rows_v[t, pl.ds(g * D + h * L, L)]
                return carry

            lax.fori_loop(0, C // G, unpack_body, 0)
            pltpu.sync_copy(out_v, out_hbm.at[pl.ds(off, C)])
            return carry

        lax.fori_loop(0, n_chunks, chunk_body, 0)

    return pack_kernel


def kernel(features, point_ids):
    M, D = features.shape
    N = point_ids.shape[0]
    y = _build_gather(N, M, D)(features, point_ids)
    y128 = y.reshape(N * D // 128, 128)
    out = _build_pack(N, D)(y128)
    return jax.lax.optimization_barrier(out)


# speedup vs baseline: 1.3206x; 1.3206x over previous
import functools

import jax
import jax.numpy as jnp
from jax import lax
from jax.experimental import pallas as pl
from jax.experimental.pallas import tpu as pltpu
from jax.experimental.pallas import tpu_sc as plsc

_CHUNK = 512


@functools.lru_cache(maxsize=None)
def _build_gather(N, M, D):
    info = plsc.get_sparse_core_info()
    num_workers = info.num_cores * info.num_subcores
    rows_per_w = N // num_workers
    C = min(_CHUNK, rows_per_w)
    n_chunks = rows_per_w // C
    assert n_chunks * C == rows_per_w and n_chunks % 2 == 0

    mesh = plsc.VectorSubcoreMesh(core_axis_name="c", subcore_axis_name="s")

    @functools.partial(
        pl.kernel,
        mesh=mesh,
        out_type=jax.ShapeDtypeStruct((N, D), jnp.float32),
        scratch_types=[
            pltpu.VMEM((2, C), jnp.int32),
            pltpu.VMEM((2, C, D), jnp.float32),
            pltpu.SemaphoreType.DMA((2,)),
            pltpu.SemaphoreType.DMA((2,)),
        ],
        compiler_params=pltpu.CompilerParams(use_tc_tiling_on_sc=False),
    )
    def gather_kernel(tbl_hbm, ids_hbm, out_hbm, idx_v, rows_v, gsem, isem):
        wid = lax.axis_index("s") * info.num_cores + lax.axis_index("c")
        base = wid * rows_per_w

        pltpu.sync_copy(ids_hbm.at[pl.ds(base, C)], idx_v.at[0])
        g0 = pltpu.async_copy(tbl_hbm.at[idx_v.at[0]], rows_v.at[0], gsem.at[0])
        pltpu.async_copy(ids_hbm.at[pl.ds(base + C, C)], idx_v.at[1], isem.at[1])

        def pair_body(t, carry):
            for b in (0, 1):
                j = t * 2 + b
                nb = 1 - b
                pltpu.make_async_copy(out_hbm.at[pl.ds(base, C)], rows_v.at[b],
                                      gsem.at[b]).wait()

                @pl.when(j + 2 < n_chunks)
                def _():
                    off2 = base + (j + 2) * C
                    pltpu.async_copy(ids_hbm.at[pl.ds(off2, C)], idx_v.at[b],
                                     isem.at[b])

                @pl.when(j + 1 < n_chunks)
                def _():
                    pltpu.make_async_copy(
                        ids_hbm.at[pl.ds(base, C)], idx_v.at[nb], isem.at[nb]
                    ).wait()
                    pltpu.async_copy(tbl_hbm.at[idx_v.at[nb]], rows_v.at[nb],
                                     gsem.at[nb])

                pltpu.sync_copy(rows_v.at[b], out_hbm.at[pl.ds(base + j * C, C)])
            return carry

        lax.fori_loop(0, n_chunks // 2, pair_body, 0)

    return gather_kernel


@functools.lru_cache(maxsize=None)
def _build_pack(N, D):
    info = plsc.get_sparse_core_info()
    L = info.num_lanes
    G = 128 // D
    num_workers = info.num_cores * info.num_subcores
    rows_per_w = N // num_workers
    C = min(_CHUNK // 2, rows_per_w)
    n_chunks = rows_per_w // C
    assert n_chunks * C == rows_per_w and n_chunks % 2 == 0

    mesh = plsc.VectorSubcoreMesh(core_axis_name="c", subcore_axis_name="s")

    @functools.partial(
        pl.kernel,
        mesh=mesh,
        out_type=jax.ShapeDtypeStruct((N, D), jnp.float32),
        scratch_types=[
            pltpu.VMEM((2, C // G, 128), jnp.float32),
            pltpu.VMEM((2, C, D), jnp.float32),
            pltpu.SemaphoreType.DMA((2,)),
            pltpu.SemaphoreType.DMA((2,)),
        ],
    )
    def pack_kernel(y_hbm, out_hbm, rows_v, out_v, rsem, wsem):
        wid = lax.axis_index("s") * info.num_cores + lax.axis_index("c")
        base = wid * rows_per_w
        base4 = wid * (rows_per_w // G)
        C4 = C // G

        pltpu.async_copy(y_hbm.at[pl.ds(base4, C4)], rows_v.at[0], rsem.at[0])
        pltpu.async_copy(y_hbm.at[pl.ds(base4 + C4, C4)], rows_v.at[1], rsem.at[1])

        def pair_body(t, carry):
            for b in (0, 1):
                j = t * 2 + b
                pltpu.make_async_copy(y_hbm.at[pl.ds(base4, C4)], rows_v.at[b],
                                      rsem.at[b]).wait()

                @pl.when(j >= 2)
                def _():
                    pltpu.make_async_copy(
                        out_v.at[b], out_hbm.at[pl.ds(base, C)], wsem.at[b]
                    ).wait()

                def unpack_body(u, carry):
                    for g in range(G):
                        for h in range(D // L):
                            out_v[b, u * G + g, pl.ds(h * L, L)] = (
                                rows_v[b, u, pl.ds(g * D + h * L, L)])
                    return carry

                lax.fori_loop(0, C4, unpack_body, 0, unroll=8)
                pltpu.async_copy(out_v.at[b],
                                 out_hbm.at[pl.ds(base + j * C, C)], wsem.at[b])

                @pl.when(j + 2 < n_chunks)
                def _():
                    off4 = base4 + (j + 2) * C4
                    pltpu.async_copy(y_hbm.at[pl.ds(off4, C4)], rows_v.at[b],
                                     rsem.at[b])
            return carry

        lax.fori_loop(0, n_chunks // 2, pair_body, 0)
        for b in (0, 1):
            pltpu.make_async_copy(out_v.at[b], out_hbm.at[pl.ds(base, C)],
                                  wsem.at[b]).wait()

    return pack_kernel


def kernel(features, point_ids):
    M, D = features.shape
    N = point_ids.shape[0]
    y = _build_gather(N, M, D)(features, point_ids)
    y128 = y.reshape(N * D // 128, 128)
    out = _build_pack(N, D)(y128)
    return jax.lax.optimization_barrier(out)


# speedup vs baseline: 1.3275x; 1.0052x over previous
import functools

import jax
import jax.numpy as jnp
from jax import lax
from jax.experimental import pallas as pl
from jax.experimental.pallas import tpu as pltpu
from jax.experimental.pallas import tpu_sc as plsc

_CHUNK = 512


@functools.lru_cache(maxsize=None)
def _build_gather(N, M, D):
    info = plsc.get_sparse_core_info()
    num_workers = info.num_cores * info.num_subcores
    rows_per_w = N // num_workers
    C = min(_CHUNK, rows_per_w)
    n_chunks = rows_per_w // C
    assert n_chunks * C == rows_per_w and n_chunks % 2 == 0

    mesh = plsc.VectorSubcoreMesh(core_axis_name="c", subcore_axis_name="s")

    @functools.partial(
        pl.kernel,
        mesh=mesh,
        out_type=jax.ShapeDtypeStruct((N, D), jnp.float32),
        scratch_types=[
            pltpu.VMEM((2, C), jnp.int32),
            pltpu.VMEM((2, C, D), jnp.float32),
            pltpu.SemaphoreType.DMA((2,)),
            pltpu.SemaphoreType.DMA((2,)),
        ],
        compiler_params=pltpu.CompilerParams(use_tc_tiling_on_sc=False),
    )
    def gather_kernel(tbl_hbm, ids_hbm, out_hbm, idx_v, rows_v, gsem, isem):
        wid = lax.axis_index("s") * info.num_cores + lax.axis_index("c")
        base = wid * rows_per_w

        pltpu.sync_copy(ids_hbm.at[pl.ds(base, C)], idx_v.at[0])
        g0 = pltpu.async_copy(tbl_hbm.at[idx_v.at[0]], rows_v.at[0], gsem.at[0])
        pltpu.async_copy(ids_hbm.at[pl.ds(base + C, C)], idx_v.at[1], isem.at[1])

        def pair_body(t, carry):
            for b in (0, 1):
                j = t * 2 + b
                nb = 1 - b
                pltpu.make_async_copy(out_hbm.at[pl.ds(base, C)], rows_v.at[b],
                                      gsem.at[b]).wait()

                @pl.when(j + 2 < n_chunks)
                def _():
                    off2 = base + (j + 2) * C
                    pltpu.async_copy(ids_hbm.at[pl.ds(off2, C)], idx_v.at[b],
                                     isem.at[b])

                @pl.when(j + 1 < n_chunks)
                def _():
                    pltpu.make_async_copy(
                        ids_hbm.at[pl.ds(base, C)], idx_v.at[nb], isem.at[nb]
                    ).wait()
                    pltpu.async_copy(tbl_hbm.at[idx_v.at[nb]], rows_v.at[nb],
                                     gsem.at[nb])

                pltpu.sync_copy(rows_v.at[b], out_hbm.at[pl.ds(base + j * C, C)])
            return carry

        lax.fori_loop(0, n_chunks // 2, pair_body, 0)

    return gather_kernel


@functools.lru_cache(maxsize=None)
def _build_droppad(M, D):
    info = plsc.get_sparse_core_info()
    L = info.num_lanes
    G = 128 // D
    num_workers = info.num_cores * info.num_subcores
    rows_per_w = M // num_workers
    C = 256
    n_chunks = rows_per_w // C
    assert n_chunks * C == rows_per_w and n_chunks % 2 == 0

    mesh = plsc.VectorSubcoreMesh(core_axis_name="c", subcore_axis_name="s")

    @functools.partial(
        pl.kernel,
        mesh=mesh,
        out_type=jax.ShapeDtypeStruct((M * D // 128, 128), jnp.float32),
        scratch_types=[
            pltpu.VMEM((2, C, D), jnp.float32),
            pltpu.VMEM((2, C // G, 128), jnp.float32),
            pltpu.SemaphoreType.DMA((2,)),
            pltpu.SemaphoreType.DMA((2,)),
        ],
    )
    def droppad_kernel(ftab_hbm, out_hbm, rows_v, out_v, rsem, wsem):
        wid = lax.axis_index("s") * info.num_cores + lax.axis_index("c")
        base = wid * rows_per_w
        base4 = wid * (rows_per_w // G)
        C4 = C // G

        pltpu.async_copy(ftab_hbm.at[pl.ds(base, C)], rows_v.at[0], rsem.at[0])
        pltpu.async_copy(ftab_hbm.at[pl.ds(base + C, C)], rows_v.at[1], rsem.at[1])

        def pair_body(t, carry):
            for b in (0, 1):
                j = t * 2 + b
                pltpu.make_async_copy(ftab_hbm.at[pl.ds(base, C)], rows_v.at[b],
                                      rsem.at[b]).wait()

                @pl.when(j >= 2)
                def _():
                    pltpu.make_async_copy(
                        out_v.at[b], out_hbm.at[pl.ds(base4, C4)], wsem.at[b]
                    ).wait()

                for u in range(C4):
                    for g in range(G):
                        for h in range(D // L):
                            out_v[b, u, pl.ds(g * D + h * L, L)] = (
                                rows_v[b, u * G + g, pl.ds(h * L, L)])

                pltpu.async_copy(out_v.at[b],
                                 out_hbm.at[pl.ds(base4 + j * C4, C4)], wsem.at[b])

                @pl.when(j + 2 < n_chunks)
                def _():
                    off = base + (j + 2) * C
                    pltpu.async_copy(ftab_hbm.at[pl.ds(off, C)], rows_v.at[b],
                                     rsem.at[b])
            return carry

        lax.fori_loop(0, n_chunks // 2, pair_body, 0)
        for b in (0, 1):
            pltpu.make_async_copy(out_v.at[b], out_hbm.at[pl.ds(base4, C4)],
                                  wsem.at[b]).wait()

    return droppad_kernel


@functools.lru_cache(maxsize=None)
def _build_pack(N, D):
    info = plsc.get_sparse_core_info()
    L = info.num_lanes
    G = 128 // D
    num_workers = info.num_cores * info.num_subcores
    rows_per_w = N // num_workers
    C = min(_CHUNK // 2, rows_per_w)
    n_chunks = rows_per_w // C
    assert n_chunks * C == rows_per_w and n_chunks % 2 == 0

    mesh = plsc.VectorSubcoreMesh(core_axis_name="c", subcore_axis_name="s")

    @functools.partial(
        pl.kernel,
        mesh=mesh,
        out_type=jax.ShapeDtypeStruct((N, D), jnp.float32),
        scratch_types=[
            pltpu.VMEM((2, C // G, 128), jnp.float32),
            pltpu.VMEM((2, C, D), jnp.float32),
            pltpu.SemaphoreType.DMA((2,)),
            pltpu.SemaphoreType.DMA((2,)),
        ],
    )
    def pack_kernel(y_hbm, out_hbm, rows_v, out_v, rsem, wsem):
        wid = lax.axis_index("s") * info.num_cores + lax.axis_index("c")
        base = wid * rows_per_w
        base4 = wid * (rows_per_w // G)
        C4 = C // G

        pltpu.async_copy(y_hbm.at[pl.ds(base4, C4)], rows_v.at[0], rsem.at[0])
        pltpu.async_copy(y_hbm.at[pl.ds(base4 + C4, C4)], rows_v.at[1], rsem.at[1])

        def pair_body(t, carry):
            for b in (0, 1):
                j = t * 2 + b
                pltpu.make_async_copy(y_hbm.at[pl.ds(base4, C4)], rows_v.at[b],
                                      rsem.at[b]).wait()

                @pl.when(j >= 2)
                def _():
                    pltpu.make_async_copy(
                        out_v.at[b], out_hbm.at[pl.ds(base, C)], wsem.at[b]
                    ).wait()

                for u in range(C4):
                    for g in range(G):
                        for h in range(D // L):
                            out_v[b, u * G + g, pl.ds(h * L, L)] = (
                                rows_v[b, u, pl.ds(g * D + h * L, L)])
                pltpu.async_copy(out_v.at[b],
                                 out_hbm.at[pl.ds(base + j * C, C)], wsem.at[b])

                @pl.when(j + 2 < n_chunks)
                def _():
                    off4 = base4 + (j + 2) * C4
                    pltpu.async_copy(y_hbm.at[pl.ds(off4, C4)], rows_v.at[b],
                                     rsem.at[b])
            return carry

        lax.fori_loop(0, n_chunks // 2, pair_body, 0)
        for b in (0, 1):
            pltpu.make_async_copy(out_v.at[b], out_hbm.at[pl.ds(base, C)],
                                  wsem.at[b]).wait()

    return pack_kernel


def kernel(features, point_ids):
    M, D = features.shape
    N = point_ids.shape[0]
    y = _build_gather(N, M, D)(features, point_ids)
    y128 = y.reshape(N * D // 128, 128)
    out = _build_pack(N, D)(y128)
    return jax.lax.optimization_barrier(out)


# speedup vs baseline: 1.3484x; 1.0158x over previous
import functools

import jax
import jax.numpy as jnp
from jax import lax
from jax.experimental import pallas as pl
from jax.experimental.pallas import tpu as pltpu
from jax.experimental.pallas import tpu_sc as plsc

_CHUNK = 1024


@functools.lru_cache(maxsize=None)
def _build_gather(N, M, D):
    info = plsc.get_sparse_core_info()
    num_workers = info.num_cores * info.num_subcores
    rows_per_w = N // num_workers
    C = min(_CHUNK, rows_per_w)
    n_chunks = rows_per_w // C
    assert n_chunks * C == rows_per_w and n_chunks % 2 == 0

    mesh = plsc.VectorSubcoreMesh(core_axis_name="c", subcore_axis_name="s")

    @functools.partial(
        pl.kernel,
        mesh=mesh,
        out_type=jax.ShapeDtypeStruct((N, D), jnp.float32),
        scratch_types=[
            pltpu.VMEM((2, C), jnp.int32),
            pltpu.VMEM((2, C, D), jnp.float32),
            pltpu.SemaphoreType.DMA((2,)),
            pltpu.SemaphoreType.DMA((2,)),
        ],
        compiler_params=pltpu.CompilerParams(use_tc_tiling_on_sc=False),
    )
    def gather_kernel(tbl_hbm, ids_hbm, out_hbm, idx_v, rows_v, gsem, isem):
        wid = lax.axis_index("s") * info.num_cores + lax.axis_index("c")
        base = wid * rows_per_w

        pltpu.sync_copy(ids_hbm.at[pl.ds(base, C)], idx_v.at[0])
        g0 = pltpu.async_copy(tbl_hbm.at[idx_v.at[0]], rows_v.at[0], gsem.at[0])
        pltpu.async_copy(ids_hbm.at[pl.ds(base + C, C)], idx_v.at[1], isem.at[1])

        def pair_body(t, carry):
            for b in (0, 1):
                j = t * 2 + b
                nb = 1 - b
                pltpu.make_async_copy(out_hbm.at[pl.ds(base, C)], rows_v.at[b],
                                      gsem.at[b]).wait()

                @pl.when(j + 2 < n_chunks)
                def _():
                    off2 = base + (j + 2) * C
                    pltpu.async_copy(ids_hbm.at[pl.ds(off2, C)], idx_v.at[b],
                                     isem.at[b])

                @pl.when(j + 1 < n_chunks)
                def _():
                    pltpu.make_async_copy(
                        ids_hbm.at[pl.ds(base, C)], idx_v.at[nb], isem.at[nb]
                    ).wait()
                    pltpu.async_copy(tbl_hbm.at[idx_v.at[nb]], rows_v.at[nb],
                                     gsem.at[nb])

                pltpu.sync_copy(rows_v.at[b], out_hbm.at[pl.ds(base + j * C, C)])
            return carry

        lax.fori_loop(0, n_chunks // 2, pair_body, 0)

    return gather_kernel


@functools.lru_cache(maxsize=None)
def _build_droppad(M, D):
    info = plsc.get_sparse_core_info()
    L = info.num_lanes
    G = 128 // D
    num_workers = info.num_cores * info.num_subcores
    rows_per_w = M // num_workers
    C = 256
    n_chunks = rows_per_w // C
    assert n_chunks * C == rows_per_w and n_chunks % 2 == 0

    mesh = plsc.VectorSubcoreMesh(core_axis_name="c", subcore_axis_name="s")

    @functools.partial(
        pl.kernel,
        mesh=mesh,
        out_type=jax.ShapeDtypeStruct((M * D // 128, 128), jnp.float32),
        scratch_types=[
            pltpu.VMEM((2, C, D), jnp.float32),
            pltpu.VMEM((2, C // G, 128), jnp.float32),
            pltpu.SemaphoreType.DMA((2,)),
            pltpu.SemaphoreType.DMA((2,)),
        ],
    )
    def droppad_kernel(ftab_hbm, out_hbm, rows_v, out_v, rsem, wsem):
        wid = lax.axis_index("s") * info.num_cores + lax.axis_index("c")
        base = wid * rows_per_w
        base4 = wid * (rows_per_w // G)
        C4 = C // G

        pltpu.async_copy(ftab_hbm.at[pl.ds(base, C)], rows_v.at[0], rsem.at[0])
        pltpu.async_copy(ftab_hbm.at[pl.ds(base + C, C)], rows_v.at[1], rsem.at[1])

        def pair_body(t, carry):
            for b in (0, 1):
                j = t * 2 + b
                pltpu.make_async_copy(ftab_hbm.at[pl.ds(base, C)], rows_v.at[b],
                                      rsem.at[b]).wait()

                @pl.when(j >= 2)
                def _():
                    pltpu.make_async_copy(
                        out_v.at[b], out_hbm.at[pl.ds(base4, C4)], wsem.at[b]
                    ).wait()

                for u in range(C4):
                    for g in range(G):
                        for h in range(D // L):
                            out_v[b, u, pl.ds(g * D + h * L, L)] = (
                                rows_v[b, u * G + g, pl.ds(h * L, L)])

                pltpu.async_copy(out_v.at[b],
                                 out_hbm.at[pl.ds(base4 + j * C4, C4)], wsem.at[b])

                @pl.when(j + 2 < n_chunks)
                def _():
                    off = base + (j + 2) * C
                    pltpu.async_copy(ftab_hbm.at[pl.ds(off, C)], rows_v.at[b],
                                     rsem.at[b])
            return carry

        lax.fori_loop(0, n_chunks // 2, pair_body, 0)
        for b in (0, 1):
            pltpu.make_async_copy(out_v.at[b], out_hbm.at[pl.ds(base4, C4)],
                                  wsem.at[b]).wait()

    return droppad_kernel


@functools.lru_cache(maxsize=None)
def _build_pack(N, D):
    info = plsc.get_sparse_core_info()
    L = info.num_lanes
    G = 128 // D
    num_workers = info.num_cores * info.num_subcores
    rows_per_w = N // num_workers
    C = min(_CHUNK // 2, rows_per_w)
    n_chunks = rows_per_w // C
    assert n_chunks * C == rows_per_w and n_chunks % 2 == 0

    mesh = plsc.VectorSubcoreMesh(core_axis_name="c", subcore_axis_name="s")

    @functools.partial(
        pl.kernel,
        mesh=mesh,
        out_type=jax.ShapeDtypeStruct((N, D), jnp.float32),
        scratch_types=[
            pltpu.VMEM((2, C // G, 128), jnp.float32),
            pltpu.VMEM((C, D), jnp.float32),
            pltpu.SemaphoreType.DMA((2,)),
            pltpu.SemaphoreType.DMA,
        ],
    )
    def pack_kernel(y_hbm, out_hbm, rows_v, out_v, rsem, wsem):
        wid = lax.axis_index("s") * info.num_cores + lax.axis_index("c")
        base = wid * rows_per_w
        base4 = wid * (rows_per_w // G)
        C4 = C // G

        pltpu.async_copy(y_hbm.at[pl.ds(base4, C4)], rows_v.at[0], rsem.at[0])
        pltpu.async_copy(y_hbm.at[pl.ds(base4 + C4, C4)], rows_v.at[1], rsem.at[1])

        def pair_body(t, carry):
            for b in (0, 1):
                j = t * 2 + b
                pltpu.make_async_copy(y_hbm.at[pl.ds(base4, C4)], rows_v.at[b],
                                      rsem.at[b]).wait()

                @pl.when(j >= 1)
                def _():
                    pltpu.make_async_copy(
                        out_v, out_hbm.at[pl.ds(base, C)], wsem
                    ).wait()

                for u in range(C4):
                    for g in range(G):
                        for h in range(D // L):
                            out_v[u * G + g, pl.ds(h * L, L)] = (
                                rows_v[b, u, pl.ds(g * D + h * L, L)])
                pltpu.async_copy(out_v,
                                 out_hbm.at[pl.ds(base + j * C, C)], wsem)

                @pl.when(j + 2 < n_chunks)
                def _():
                    off4 = base4 + (j + 2) * C4
                    pltpu.async_copy(y_hbm.at[pl.ds(off4, C4)], rows_v.at[b],
                                     rsem.at[b])
            return carry

        lax.fori_loop(0, n_chunks // 2, pair_body, 0)
        pltpu.make_async_copy(out_v, out_hbm.at[pl.ds(base, C)], wsem).wait()

    return pack_kernel


def kernel(features, point_ids):
    M, D = features.shape
    N = point_ids.shape[0]
    y = _build_gather(N, M, D)(features, point_ids)
    y128 = y.reshape(N * D // 128, 128)
    out = _build_pack(N, D)(y128)
    return jax.lax.optimization_barrier(out)
